# baseline (speedup 1.0000x reference)
import jax
import jax.numpy as jnp
from jax import lax
from jax.experimental import pallas as pl
from jax.experimental.pallas import tpu as pltpu
from jax.experimental.pallas import tpu_sc as plsc

VOCAB = 1000000
CTX = 200
DIM = 32
BATCH = 4096

NC = 2
NS = 16
NW = NC * NS
ROWS_W = BATCH // NW
NB = 8
G = ROWS_W // NB
CH = NB * CTX


TCOLS = 999936
TBLK = 512
NBLK = TCOLS // TBLK
NLOC_T = -(-NBLK // NW)
NLOC_T += NLOC_T % 2


def _transpose_body(tokT_hbm, tail_hbm, tok128_hbm,
                    i0_v, i1_v, o0_v, o1_v, isem0, isem1, osem0, osem1):
    wid = lax.axis_index("s") * NC + lax.axis_index("c")
    ibufs = (i0_v, i1_v)
    obufs = (o0_v, o1_v)
    isems = (isem0, isem1)
    osems = (osem0, osem1)

    @pl.when(wid == 0)
    def _():
        pltpu.sync_copy(tail_hbm, o0_v.at[pl.ds(0, 16)])
        pltpu.sync_copy(o0_v.at[pl.ds(0, 16)],
                        tok128_hbm.at[pl.ds(TCOLS // 4, 16), :])

    def blk(t):
        return lax.rem(t * NW + wid, NBLK)

    def in_copy(c, buf):
        return pltpu.make_async_copy(
            tokT_hbm.at[:, pl.ds(c * TBLK, TBLK)],
            ibufs[buf].at[:, pl.ds(0, TBLK)], isems[buf])

    def out_copy(c, buf):
        return pltpu.make_async_copy(
            obufs[buf], tok128_hbm.at[pl.ds(c * (TBLK // 4), TBLK // 4), :],
            osems[buf])

    iota = lax.iota(jnp.int32, 16)
    iota_hi = iota + 16

    def transpose(buf):
        inb = ibufs[buf]
        outb = obufs[buf]

        def grp(q, _):
            for j in range(16):
                t = q * 16 + j
                tv = iota * 0 + t
                lo = plsc.load_gather(inb, [iota, tv])
                hi = plsc.load_gather(inb, [iota_hi, tv])
                r = q * 4 + j // 4
                c0 = (j % 4) * DIM
                outb[r, pl.ds(c0, 16)] = lo
                outb[r, pl.ds(c0 + 16, 16)] = hi
            return 0

        lax.fori_loop(0, TBLK // 16, grp, 0)

    def phase(t, cur, oth, first, last):
        if not first:
            out_copy(blk(t - 1), oth).wait()
        if not last:
            in_copy(blk(t + 1), oth).start()
        in_copy(blk(t), cur).wait()
        transpose(cur)
        out_copy(blk(t), cur).start()

    in_copy(blk(0), 0).start()
    phase(0, 0, 1, first=True, last=False)
    phase(1, 1, 0, first=False, last=False)

    def pairs(k, _):
        t = 2 * k
        phase(t, 0, 1, first=False, last=False)
        phase(t + 1, 1, 0, first=False, last=False)
        return 0

    lax.fori_loop(1, NLOC_T // 2 - 1, pairs, 0)

    phase(NLOC_T - 2, 0, 1, first=False, last=False)
    phase(NLOC_T - 1, 1, 0, first=False, last=True)
    out_copy(blk(NLOC_T - 1), 1).wait()


@jax.jit
def _transpose(tokT, tail128):
    mesh = plsc.VectorSubcoreMesh(core_axis_name="c", subcore_axis_name="s")
    return pl.kernel(
        _transpose_body,
        out_type=jax.ShapeDtypeStruct((VOCAB // 4, 128), jnp.float32),
        mesh=mesh,
        scratch_types=[
            pltpu.VMEM((DIM, TBLK + 1), jnp.float32),
            pltpu.VMEM((DIM, TBLK + 1), jnp.float32),
            pltpu.VMEM((TBLK // 4, 128), jnp.float32),
            pltpu.VMEM((TBLK // 4, 128), jnp.float32),
            pltpu.SemaphoreType.DMA,
            pltpu.SemaphoreType.DMA,
            pltpu.SemaphoreType.DMA,
            pltpu.SemaphoreType.DMA,
        ],
        compiler_params=pltpu.CompilerParams(needs_layout_passes=False),
    )(tokT, tail128)


def _body(x_hbm, tok_hbm, pos_hbm, out_hbm, idx_v, rows_v, pos_v, sem):
    wid = lax.axis_index("s") * NC + lax.axis_index("c")
    base = wid * ROWS_W

    pltpu.sync_copy(pos_hbm, pos_v)

    for g in range(G):
        b0 = base + g * NB
        pltpu.sync_copy(x_hbm.at[pl.ds(b0, NB), :], idx_v)
        for sb in range(NB):
            pltpu.async_copy(tok_hbm.at[idx_v.at[sb]], rows_v.at[sb], sem)
        for sb in range(NB):
            pltpu.make_async_copy(tok_hbm.at[idx_v.at[sb]], rows_v.at[sb], sem).wait()

        def add_l(l, _):
            p0 = pos_v[l, pl.ds(0, 16)]
            p1 = pos_v[l, pl.ds(16, 16)]
            for sb in range(NB):
                rows_v[sb, l, pl.ds(0, 16)] = rows_v[sb, l, pl.ds(0, 16)] + p0
                rows_v[sb, l, pl.ds(16, 16)] = rows_v[sb, l, pl.ds(16, 16)] + p1
            return 0

        lax.fori_loop(0, CTX, add_l, 0, unroll=2)

        pltpu.sync_copy(rows_v, out_hbm.at[pl.ds(b0, NB), :, :])


@jax.jit
def _embed(x, token_table, pos_table):
    mesh = plsc.VectorSubcoreMesh(core_axis_name="c", subcore_axis_name="s")
    return pl.kernel(
        _body,
        out_type=jax.ShapeDtypeStruct((BATCH, CTX, DIM), jnp.float32),
        mesh=mesh,
        scratch_types=[
            pltpu.VMEM((NB, CTX), jnp.int32),
            pltpu.VMEM((NB, CTX, DIM), jnp.float32),
            pltpu.VMEM((CTX, DIM), jnp.float32),
            pltpu.SemaphoreType.DMA,
        ],
        compiler_params=pltpu.CompilerParams(use_tc_tiling_on_sc=False),
    )(x, token_table, pos_table)


def kernel(x, token_table, pos_table):
    tail128 = jnp.reshape(token_table[TCOLS:, :], (16, 128))
    tok128 = _transpose(token_table.T, tail128)
    tok_lin = jnp.reshape(tok128, (VOCAB, DIM))
    return _embed(x.astype(jnp.int32), tok_lin, pos_table)


# speedup vs baseline: 1.6395x; 1.6395x over previous
import jax
import jax.numpy as jnp
from jax import lax
from jax.experimental import pallas as pl
from jax.experimental.pallas import tpu as pltpu
from jax.experimental.pallas import tpu_sc as plsc

VOCAB = 1000000
CTX = 200
DIM = 32
BATCH = 4096

NC = 2
NS = 16
NW = NC * NS
ROWS_W = BATCH // NW
NB = 8
G = ROWS_W // NB
CH = NB * CTX


def _body(x_hbm, tok_hbm, pos_hbm, out_hbm, idx_v, rows_v, pos_v, sem):
    wid = lax.axis_index("s") * NC + lax.axis_index("c")
    base = wid * ROWS_W

    pltpu.sync_copy(pos_hbm, pos_v)

    for g in range(G):
        b0 = base + g * NB
        pltpu.sync_copy(x_hbm.at[pl.ds(b0, NB), :], idx_v)
        for sb in range(NB):
            pltpu.async_copy(tok_hbm.at[idx_v.at[sb]], rows_v.at[sb], sem)
        for sb in range(NB):
            pltpu.make_async_copy(tok_hbm.at[idx_v.at[sb]], rows_v.at[sb], sem).wait()

        def add_l(l, _):
            p0 = pos_v[l, pl.ds(0, 16)]
            p1 = pos_v[l, pl.ds(16, 16)]
            for sb in range(NB):
                rows_v[sb, l, pl.ds(0, 16)] = rows_v[sb, l, pl.ds(0, 16)] + p0
                rows_v[sb, l, pl.ds(16, 16)] = rows_v[sb, l, pl.ds(16, 16)] + p1
            return 0

        lax.fori_loop(0, CTX, add_l, 0, unroll=2)

        pltpu.sync_copy(rows_v, out_hbm.at[pl.ds(b0, NB), :, pl.ds(0, DIM)])


@jax.jit
def _embed(x, token_table, pos_table):
    mesh = plsc.VectorSubcoreMesh(core_axis_name="c", subcore_axis_name="s")
    padded = pl.kernel(
        _body,
        out_type=jax.ShapeDtypeStruct((BATCH, CTX, 128), jnp.float32),
        mesh=mesh,
        scratch_types=[
            pltpu.VMEM((NB, CTX), jnp.int32),
            pltpu.VMEM((NB, CTX, DIM), jnp.float32),
            pltpu.VMEM((CTX, DIM), jnp.float32),
            pltpu.SemaphoreType.DMA,
        ],
        compiler_params=pltpu.CompilerParams(use_tc_tiling_on_sc=False),
    )(x, token_table, pos_table)
    return lax.slice(padded, (0, 0, 0), (BATCH, CTX, DIM))


def kernel(x, token_table, pos_table):
    return _embed(x.astype(jnp.int32), token_table, pos_table)
